# baseline (speedup 1.0000x reference)
import functools

import jax
import jax.numpy as jnp
from jax import lax
from jax.experimental import pallas as pl
from jax.experimental.pallas import tpu as pltpu
from jax.experimental.pallas import tpu_sc as plsc

V = 1000000
EMB = 64
SCALE = 8.0
LANES = 16

NC = 2
NS = 16
NW = NC * NS

TCHUNK = 128
NKFULL = 122
NCHUNKS = (V // (2 * TCHUNK)) * 0 + 3906
VTAIL = NCHUNKS * 2 * TCHUNK
NTAILP = (V - VTAIL) // 2

CHUNK = 128
NBUF = 4

_mesh = lambda: plsc.VectorSubcoreMesh(core_axis_name="c", subcore_axis_name="s")
_params = lambda: pltpu.CompilerParams(use_tc_tiling_on_sc=True, needs_layout_passes=False)


def _make_transpose():
    @functools.partial(
        pl.kernel,
        out_type=jax.ShapeDtypeStruct((V // 2, 2 * EMB), jnp.float32),
        mesh=_mesh(),
        compiler_params=_params(),
        scratch_types=(
            [pltpu.VMEM((EMB, 2 * TCHUNK), jnp.float32) for _ in range(2)]
            + [pltpu.VMEM((TCHUNK, 2 * EMB + 1), jnp.float32) for _ in range(2)]
            + [pltpu.SemaphoreType.DMA for _ in range(4)]
        ),
    )
    def t_kernel(tabt_hbm, scr_hbm, tin0, tin1, tout0, tout1, si0, si1, so0, so1):
        tins, touts = (tin0, tin1), (tout0, tout1)
        sins, souts = (si0, si1), (so0, so1)
        wid = lax.axis_index("s") * NC + lax.axis_index("c")
        nk = jnp.where(wid < NCHUNKS - NW * NKFULL, NKFULL + 1, NKFULL)
        iota = lax.iota(jnp.int32, LANES)
        half = lax.shift_right_logical(iota, 1)
        par64 = (iota & 1) * EMB
        rowbases = [half + x * (LANES // 2) for x in range(LANES)]

        def cof(k):
            return (wid + NW * k) * (2 * TCHUNK)

        def rof(k):
            return (wid + NW * k) * TCHUNK

        def in_start(k, b):
            pltpu.async_copy(tabt_hbm.at[:, pl.ds(cof(k), 2 * TCHUNK)], tins[b], sins[b])

        def in_wait(k, b):
            pltpu.make_async_copy(tabt_hbm.at[:, pl.ds(cof(k), 2 * TCHUNK)], tins[b], sins[b]).wait()

        def out_start(k, b):
            pltpu.async_copy(touts[b].at[:, pl.ds(0, 2 * EMB)],
                             scr_hbm.at[pl.ds(rof(k), TCHUNK)], souts[b])

        def out_wait(k, b):
            pltpu.make_async_copy(touts[b].at[:, pl.ds(0, 2 * EMB)],
                                  scr_hbm.at[pl.ds(rof(k), TCHUNK)], souts[b]).wait()

        def transpose_chunk(b):
            def dloop(d, carry, b=b):
                colv = par64 + d
                vals = [tins[b][d, pl.ds(x * LANES, LANES)] for x in range(LANES)]
                for x in range(LANES):
                    plsc.store_scatter(touts[b], [rowbases[x], colv], vals[x])
                return carry
            lax.fori_loop(0, EMB, dloop, 0)

        in_start(0, 0)
        in_start(1, 1)

        def body(kk, carry):
            for b in (0, 1):
                k = kk * 2 + b
                in_wait(k, b)

                @pl.when(k >= 2)
                def _(k=k, b=b):
                    out_wait(k - 2, b)

                transpose_chunk(b)
                out_start(k, b)

                @pl.when(k + 2 < nk)
                def _(k=k, b=b):
                    in_start(k + 2, b)
            return carry

        lax.fori_loop(0, NKFULL // 2, body, 0)

        @pl.when(nk == NKFULL + 1)
        def _():
            in_wait(NKFULL, 0)
            out_wait(NKFULL - 2, 0)
            transpose_chunk(0)
            out_start(NKFULL, 0)
            out_wait(NKFULL, 0)
            out_wait(NKFULL - 1, 1)

        @pl.when(nk == NKFULL)
        def _():
            out_wait(NKFULL - 2, 0)
            out_wait(NKFULL - 1, 1)

    return t_kernel


def _make_gather(nbatch, npos):
    bblk = nbatch // CHUNK

    @functools.partial(
        pl.kernel,
        out_type=jax.ShapeDtypeStruct((npos, EMB // 8, bblk, 8, CHUNK), jnp.float32),
        mesh=_mesh(),
        compiler_params=_params(),
        scratch_types=(
            [pltpu.VMEM((npos, CHUNK), jnp.int32),
             pltpu.VMEM((NBUF, CHUNK), jnp.int32),
             pltpu.VMEM((NTAILP, 2 * EMB), jnp.float32)]
            + [pltpu.VMEM((CHUNK, 2 * EMB), jnp.float32) for _ in range(NBUF)]
            + [pltpu.VMEM((EMB // 8, 8, CHUNK + 1), jnp.float32) for _ in range(2)]
            + [pltpu.SemaphoreType.DMA for _ in range(NBUF + 2)]
        ),
    )
    def g_kernel(tok_hbm, scr_hbm, tail_hbm, out_hbm, idx_v, pidx_v, tail_v, *rest):
        gbufs = rest[:NBUF]
        obufs = rest[NBUF:NBUF + 2]
        gsems = rest[NBUF + 2:2 * NBUF + 2]
        osems = rest[2 * NBUF + 2:]

        wid = lax.axis_index("s") * NC + lax.axis_index("c")
        iota = lax.iota(jnp.int32, LANES)

        pltpu.sync_copy(tok_hbm.at[:, pl.ds(wid * CHUNK, CHUNK)], idx_v)
        pltpu.sync_copy(tail_hbm, tail_v)

        def make_pidx(s, b):
            for j in range(CHUNK // LANES):
                sl = pl.ds(j * LANES, LANES)
                pidx_v[b, sl] = lax.shift_right_logical(idx_v[s, sl], 1)

        def gather(b):
            pltpu.async_copy(scr_hbm.at[pidx_v.at[b]], gbufs[b], gsems[b])

        def gather_wait(b):
            pltpu.make_async_copy(scr_hbm.at[pidx_v.at[b]], gbufs[b], gsems[b]).wait()

        def out_start(s, ob):
            pltpu.async_copy(obufs[ob].at[:, :, pl.ds(0, CHUNK)],
                             out_hbm.at[s, :, wid], osems[ob])

        def out_wait(s, ob):
            pltpu.make_async_copy(obufs[ob].at[:, :, pl.ds(0, CHUNK)],
                                  out_hbm.at[s, :, wid], osems[ob]).wait()

        def scalar(x):
            return x[0] if x.ndim else x

        dblks = [lax.shift_right_logical(j * LANES + iota, 3) for j in range(EMB // LANES)]
        dsubs = [(j * LANES + iota) & 7 for j in range(EMB // LANES)]

        def block(s, b, ob):
            def krow(k, carry, b=b, ob=ob, s=s):
                parv = (idx_v[s, pl.ds(k * LANES, LANES)] & 1) * EMB
                base = k * LANES
                for half in range(4):
                    loaded = []
                    for q in range(4):
                        lane = half * 4 + q
                        r = base + lane
                        off = parv[lane]
                        loaded.append((r, [gbufs[b][r, pl.ds(off + j * LANES, LANES)]
                                           for j in range(EMB // LANES)]))
                    for r, vs in loaded:
                        colr = jnp.full((LANES,), r, jnp.int32)
                        for j in range(EMB // LANES):
                            plsc.store_scatter(obufs[ob], [dblks[j], dsubs[j], colr],
                                               vs[j] * SCALE)
                return carry
            lax.fori_loop(0, CHUNK // LANES, krow, 0)

            tvecs = [idx_v[s, pl.ds(g * LANES, LANES)] for g in range(CHUNK // LANES)]
            par64s = [(t & 1) * EMB for t in tvecs]
            masks = [t >= VTAIL for t in tvecs]
            cnts = [scalar(plsc.all_reduce_population_count(m)) for m in masks]
            for g in range(CHUNK // LANES):
                @pl.when(cnts[g] > 0)
                def _(g=g, ob=ob):
                    trow = lax.shift_right_logical(tvecs[g] - VTAIL, 1) & (NTAILP - 1)

                    def tloop(d, carry):
                        dblk = lax.shift_right_logical(d, 3)
                        dsub = d & 7
                        colv = par64s[g] + d
                        vt = plsc.load_gather(tail_v, [trow, colv], mask=masks[g])
                        cur = obufs[ob][dblk, dsub, pl.ds(g * LANES, LANES)]
                        obufs[ob][dblk, dsub, pl.ds(g * LANES, LANES)] = jnp.where(
                            masks[g], vt * SCALE, cur)
                        return carry
                    lax.fori_loop(0, EMB, tloop, 0)

        for b in range(NBUF):
            make_pidx(b, b)
            gather(b)

        def outer(it, carry):
            for b in range(NBUF):
                s = it * NBUF + b
                ob = b % 2
                gather_wait(b)

                bt = (b + 2) % NBUF
                @pl.when(jnp.logical_and(s >= 2, s <= npos - 3))
                def _(s=s, bt=bt):
                    make_pidx(s + 2, bt)
                    gather(bt)

                @pl.when(s >= 2)
                def _(s=s, ob=ob):
                    out_wait(s - 2, ob)

                block(s, b, ob)
                out_start(s, ob)
            return carry

        lax.fori_loop(0, npos // NBUF, outer, 0)

        out_wait(npos - 2, 0)
        out_wait(npos - 1, 1)

    return g_kernel


def kernel(tokens, table):
    nbatch, npos = tokens.shape
    tabt = table.T
    tokt = tokens.T.astype(jnp.int32)
    scratch = _make_transpose()(tabt)
    tail = table[VTAIL:].reshape(NTAILP, 2 * EMB)
    out5 = _make_gather(nbatch, npos)(tokt, scratch, tail)
    return out5.transpose(2, 4, 0, 1, 3).reshape(nbatch, npos, EMB)


# speedup vs baseline: 1.0019x; 1.0019x over previous
import functools

import jax
import jax.numpy as jnp
from jax import lax
from jax.experimental import pallas as pl
from jax.experimental.pallas import tpu as pltpu
from jax.experimental.pallas import tpu_sc as plsc

V = 1000000
EMB = 64
SCALE = 8.0
LANES = 16

NC = 2
NS = 16
NW = NC * NS

TCHUNK = 128
NKFULL = 122
NCHUNKS = (V // (2 * TCHUNK)) * 0 + 3906
VTAIL = NCHUNKS * 2 * TCHUNK
NTAILP = (V - VTAIL) // 2

CHUNK = 128
NBUF = 4

_mesh = lambda: plsc.VectorSubcoreMesh(core_axis_name="c", subcore_axis_name="s")
_params = lambda: pltpu.CompilerParams(use_tc_tiling_on_sc=True, needs_layout_passes=False)


def _make_transpose():
    @functools.partial(
        pl.kernel,
        out_type=jax.ShapeDtypeStruct((V // 2, 2 * EMB), jnp.float32),
        mesh=_mesh(),
        compiler_params=_params(),
        scratch_types=(
            [pltpu.VMEM((EMB, 2 * TCHUNK), jnp.float32) for _ in range(2)]
            + [pltpu.VMEM((TCHUNK, 2 * EMB + 8), jnp.float32) for _ in range(2)]
            + [pltpu.SemaphoreType.DMA for _ in range(4)]
        ),
    )
    def t_kernel(tabt_hbm, scr_hbm, tin0, tin1, tout0, tout1, si0, si1, so0, so1):
        tins, touts = (tin0, tin1), (tout0, tout1)
        sins, souts = (si0, si1), (so0, so1)
        wid = lax.axis_index("s") * NC + lax.axis_index("c")
        nk = jnp.where(wid < NCHUNKS - NW * NKFULL, NKFULL + 1, NKFULL)
        iota = lax.iota(jnp.int32, LANES)
        half = lax.shift_right_logical(iota, 1)
        par64 = (iota & 1) * EMB
        rowbases = [half + x * (LANES // 2) for x in range(LANES)]

        def cof(k):
            return (wid + NW * k) * (2 * TCHUNK)

        def rof(k):
            return (wid + NW * k) * TCHUNK

        def in_start(k, b):
            pltpu.async_copy(tabt_hbm.at[:, pl.ds(cof(k), 2 * TCHUNK)], tins[b], sins[b])

        def in_wait(k, b):
            pltpu.make_async_copy(tabt_hbm.at[:, pl.ds(cof(k), 2 * TCHUNK)], tins[b], sins[b]).wait()

        def out_start(k, b):
            pltpu.async_copy(touts[b].at[:, pl.ds(0, 2 * EMB)],
                             scr_hbm.at[pl.ds(rof(k), TCHUNK)], souts[b])

        def out_wait(k, b):
            pltpu.make_async_copy(touts[b].at[:, pl.ds(0, 2 * EMB)],
                                  scr_hbm.at[pl.ds(rof(k), TCHUNK)], souts[b]).wait()

        def transpose_chunk(b):
            def dloop(d, carry, b=b):
                colv = par64 + d
                vals = [tins[b][d, pl.ds(x * LANES, LANES)] for x in range(LANES)]
                for x in range(LANES):
                    plsc.store_scatter(touts[b], [rowbases[x], colv], vals[x])
                return carry
            lax.fori_loop(0, EMB, dloop, 0)

        in_start(0, 0)
        in_start(1, 1)

        def body(kk, carry):
            for b in (0, 1):
                k = kk * 2 + b
                in_wait(k, b)

                @pl.when(k >= 2)
                def _(k=k, b=b):
                    out_wait(k - 2, b)

                transpose_chunk(b)
                out_start(k, b)

                @pl.when(k + 2 < nk)
                def _(k=k, b=b):
                    in_start(k + 2, b)
            return carry

        lax.fori_loop(0, NKFULL // 2, body, 0)

        @pl.when(nk == NKFULL + 1)
        def _():
            in_wait(NKFULL, 0)
            out_wait(NKFULL - 2, 0)
            transpose_chunk(0)
            out_start(NKFULL, 0)
            out_wait(NKFULL, 0)
            out_wait(NKFULL - 1, 1)

        @pl.when(nk == NKFULL)
        def _():
            out_wait(NKFULL - 2, 0)
            out_wait(NKFULL - 1, 1)

    return t_kernel


def _make_gather(nbatch, npos):
    bblk = nbatch // CHUNK

    @functools.partial(
        pl.kernel,
        out_type=jax.ShapeDtypeStruct((npos, EMB // 8, bblk, 8, CHUNK), jnp.float32),
        mesh=_mesh(),
        compiler_params=_params(),
        scratch_types=(
            [pltpu.VMEM((npos, CHUNK), jnp.int32),
             pltpu.VMEM((NBUF, CHUNK), jnp.int32),
             pltpu.VMEM((NTAILP, 2 * EMB), jnp.float32)]
            + [pltpu.VMEM((CHUNK, 2 * EMB), jnp.float32) for _ in range(NBUF)]
            + [pltpu.VMEM((EMB // 8, 8, CHUNK + 8), jnp.float32) for _ in range(2)]
            + [pltpu.SemaphoreType.DMA for _ in range(NBUF + 2)]
        ),
    )
    def g_kernel(tok_hbm, scr_hbm, tail_hbm, out_hbm, idx_v, pidx_v, tail_v, *rest):
        gbufs = rest[:NBUF]
        obufs = rest[NBUF:NBUF + 2]
        gsems = rest[NBUF + 2:2 * NBUF + 2]
        osems = rest[2 * NBUF + 2:]

        wid = lax.axis_index("s") * NC + lax.axis_index("c")
        iota = lax.iota(jnp.int32, LANES)

        pltpu.sync_copy(tok_hbm.at[:, pl.ds(wid * CHUNK, CHUNK)], idx_v)
        pltpu.sync_copy(tail_hbm, tail_v)

        def make_pidx(s, b):
            for j in range(CHUNK // LANES):
                sl = pl.ds(j * LANES, LANES)
                pidx_v[b, sl] = lax.shift_right_logical(idx_v[s, sl], 1)

        def gather(b):
            pltpu.async_copy(scr_hbm.at[pidx_v.at[b]], gbufs[b], gsems[b])

        def gather_wait(b):
            pltpu.make_async_copy(scr_hbm.at[pidx_v.at[b]], gbufs[b], gsems[b]).wait()

        def out_start(s, ob):
            pltpu.async_copy(obufs[ob].at[:, :, pl.ds(0, CHUNK)],
                             out_hbm.at[s, :, wid], osems[ob])

        def out_wait(s, ob):
            pltpu.make_async_copy(obufs[ob].at[:, :, pl.ds(0, CHUNK)],
                                  out_hbm.at[s, :, wid], osems[ob]).wait()

        def scalar(x):
            return x[0] if x.ndim else x

        dblks = [lax.shift_right_logical(j * LANES + iota, 3) for j in range(EMB // LANES)]
        dsubs = [(j * LANES + iota) & 7 for j in range(EMB // LANES)]

        def block(s, b, ob):
            def krow(k, carry, b=b, ob=ob, s=s):
                parv = (idx_v[s, pl.ds(k * LANES, LANES)] & 1) * EMB
                base = k * LANES
                for half in range(4):
                    loaded = []
                    for q in range(4):
                        lane = half * 4 + q
                        r = base + lane
                        off = parv[lane]
                        loaded.append((r, [gbufs[b][r, pl.ds(off + j * LANES, LANES)]
                                           for j in range(EMB // LANES)]))
                    for r, vs in loaded:
                        colr = jnp.full((LANES,), r, jnp.int32)
                        for j in range(EMB // LANES):
                            plsc.store_scatter(obufs[ob], [dblks[j], dsubs[j], colr],
                                               vs[j] * SCALE)
                return carry
            lax.fori_loop(0, CHUNK // LANES, krow, 0)

            tvecs = [idx_v[s, pl.ds(g * LANES, LANES)] for g in range(CHUNK // LANES)]
            par64s = [(t & 1) * EMB for t in tvecs]
            masks = [t >= VTAIL for t in tvecs]
            cnts = [scalar(plsc.all_reduce_population_count(m)) for m in masks]
            for g in range(CHUNK // LANES):
                @pl.when(cnts[g] > 0)
                def _(g=g, ob=ob):
                    trow = lax.shift_right_logical(tvecs[g] - VTAIL, 1) & (NTAILP - 1)

                    def tloop(d, carry):
                        dblk = lax.shift_right_logical(d, 3)
                        dsub = d & 7
                        colv = par64s[g] + d
                        vt = plsc.load_gather(tail_v, [trow, colv], mask=masks[g])
                        cur = obufs[ob][dblk, dsub, pl.ds(g * LANES, LANES)]
                        obufs[ob][dblk, dsub, pl.ds(g * LANES, LANES)] = jnp.where(
                            masks[g], vt * SCALE, cur)
                        return carry
                    lax.fori_loop(0, EMB, tloop, 0)

        for b in range(NBUF):
            make_pidx(b, b)
            gather(b)

        def outer(it, carry):
            for b in range(NBUF):
                s = it * NBUF + b
                ob = b % 2
                gather_wait(b)

                bt = (b + 2) % NBUF
                @pl.when(jnp.logical_and(s >= 2, s <= npos - 3))
                def _(s=s, bt=bt):
                    make_pidx(s + 2, bt)
                    gather(bt)

                @pl.when(s >= 2)
                def _(s=s, ob=ob):
                    out_wait(s - 2, ob)

                block(s, b, ob)
                out_start(s, ob)
            return carry

        lax.fori_loop(0, npos // NBUF, outer, 0)

        out_wait(npos - 2, 0)
        out_wait(npos - 1, 1)

    return g_kernel


def kernel(tokens, table):
    nbatch, npos = tokens.shape
    tabt = table.T
    tokt = tokens.T.astype(jnp.int32)
    scratch = _make_transpose()(tabt)
    tail = table[VTAIL:].reshape(NTAILP, 2 * EMB)
    out5 = _make_gather(nbatch, npos)(tokt, scratch, tail)
    return out5.transpose(2, 4, 0, 1, 3).reshape(nbatch, npos, EMB)


# speedup vs baseline: 1.7552x; 1.7519x over previous
import functools

import jax
import jax.numpy as jnp
from jax import lax
from jax.experimental import pallas as pl
from jax.experimental.pallas import tpu as pltpu
from jax.experimental.pallas import tpu_sc as plsc

EMB = 64
SCALE = 8.0

NC = 2
NS = 16
NW = NC * NS

CHUNK = 128
NBUF = 4
NOBUF = 2
LANES = 16


def _build(num_tokens):
    per_w = num_tokens // NW
    ngroups = per_w // CHUNK
    iters = ngroups // NBUF
    mesh = plsc.VectorSubcoreMesh(core_axis_name="c", subcore_axis_name="s")

    @functools.partial(
        pl.kernel,
        out_type=jax.ShapeDtypeStruct((num_tokens, EMB), jnp.float32),
        mesh=mesh,
        compiler_params=pltpu.CompilerParams(use_tc_tiling_on_sc=True),
        scratch_types=(
            [pltpu.VMEM((ngroups, CHUNK), jnp.int32),
             pltpu.VMEM((NBUF, CHUNK), jnp.int32)]
            + [pltpu.VMEM((CHUNK, 2 * EMB), jnp.float32) for _ in range(NBUF)]
            + [pltpu.VMEM((CHUNK, EMB), jnp.float32) for _ in range(NOBUF)]
            + [pltpu.SemaphoreType.DMA for _ in range(NBUF + NOBUF)]
        ),
    )
    def emb_kernel(tokens_hbm, table_hbm, out_hbm, idx_v, pidx_v, *rest):
        gbufs = rest[:NBUF]
        obufs = rest[NBUF:NBUF + NOBUF]
        gsems = rest[NBUF + NOBUF:2 * NBUF + NOBUF]
        osems = rest[2 * NBUF + NOBUF:]

        wid = lax.axis_index("s") * NC + lax.axis_index("c")
        base = wid * per_w

        pltpu.sync_copy(tokens_hbm.at[wid], idx_v)

        def make_pidx(gi, b):
            for j in range(CHUNK // LANES):
                sl = pl.ds(j * LANES, LANES)
                pidx_v[b, sl] = lax.shift_right_logical(idx_v[gi, sl], 1)

        def gather(b):
            pltpu.async_copy(table_hbm.at[pidx_v.at[b]], gbufs[b], gsems[b])

        def gather_wait(b):
            pltpu.make_async_copy(table_hbm.at[pidx_v.at[b]], gbufs[b], gsems[b]).wait()

        def out_ref(gi):
            return out_hbm.at[pl.ds(base + gi * CHUNK, CHUNK)]

        def out_start(gi, ob):
            pltpu.async_copy(obufs[ob], out_ref(gi), osems[ob])

        def out_wait(gi, ob):
            pltpu.make_async_copy(obufs[ob], out_ref(gi), osems[ob]).wait()

        for b in range(NBUF):
            make_pidx(b, b)
            gather(b)

        def outer(it, carry):
            for b in range(NBUF):
                gi = it * NBUF + b
                ob = b % NOBUF
                gather_wait(b)

                bt = (b + 2) % NBUF
                @pl.when(jnp.logical_and(gi >= 2, gi <= ngroups - 3))
                def _(gi=gi, bt=bt):
                    make_pidx(gi + 2, bt)
                    gather(bt)

                @pl.when(gi >= NOBUF)
                def _(gi=gi, ob=ob):
                    out_wait(gi - NOBUF, ob)

                def mul_body(k, c, b=b, gi=gi, ob=ob):
                    par = (idx_v[gi, pl.ds(k * LANES, LANES)] & 1) * EMB
                    for rr in range(LANES):
                        row = k * LANES + rr
                        off = par[rr]
                        for j in range(EMB // LANES):
                            src = pl.ds(off + j * LANES, LANES)
                            dst = pl.ds(j * LANES, LANES)
                            obufs[ob][row, dst] = gbufs[b][row, src] * SCALE
                    return c
                lax.fori_loop(0, CHUNK // LANES, mul_body, 0)

                out_start(gi, ob)
            return carry

        lax.fori_loop(0, iters, outer, 0)

        for ob in range(NOBUF):
            out_wait(ngroups - NOBUF + ob, ob)

    return emb_kernel


def kernel(tokens, table):
    num_tokens = tokens.size
    tokens3 = tokens.reshape(NW, num_tokens // (NW * CHUNK), CHUNK).astype(jnp.int32)
    table2 = table.reshape(table.shape[0] // 2, 2 * EMB)
    out = _build(num_tokens)(tokens3, table2)
    return out.reshape(tokens.shape + (EMB,))


# speedup vs baseline: 1.8107x; 1.0316x over previous
import functools

import jax
import jax.numpy as jnp
from jax import lax
from jax.experimental import pallas as pl
from jax.experimental.pallas import tpu as pltpu
from jax.experimental.pallas import tpu_sc as plsc

EMB = 64
SCALE = 8.0

NC = 2
NS = 16
NW = NC * NS

CHUNK = 128
NBUF = 4
MUL_UNROLL = 8


def _build(num_tokens):
    per_w = num_tokens // NW
    ngroups = per_w // CHUNK
    iters = ngroups // NBUF
    mesh = plsc.VectorSubcoreMesh(core_axis_name="c", subcore_axis_name="s")

    @functools.partial(
        pl.kernel,
        out_type=jax.ShapeDtypeStruct((num_tokens, EMB), jnp.float32),
        mesh=mesh,
        compiler_params=pltpu.CompilerParams(use_tc_tiling_on_sc=False),
        scratch_types=(
            [pltpu.VMEM((ngroups, CHUNK), jnp.int32)]
            + [pltpu.VMEM((CHUNK, EMB), jnp.float32) for _ in range(NBUF)]
            + [pltpu.SemaphoreType.DMA for _ in range(2 * NBUF)]
        ),
    )
    def emb_kernel(tokens_hbm, table_hbm, out_hbm, idx_v, *rest):
        bufs = rest[:NBUF]
        gsems = rest[NBUF:2 * NBUF]
        osems = rest[2 * NBUF:]

        wid = lax.axis_index("s") * NC + lax.axis_index("c")
        base = wid * per_w

        pltpu.sync_copy(tokens_hbm.at[wid], idx_v)

        def gather(gi, b):
            pltpu.async_copy(table_hbm.at[idx_v.at[gi]], bufs[b], gsems[b])

        def gather_wait(gi, b):
            pltpu.make_async_copy(table_hbm.at[idx_v.at[gi]], bufs[b], gsems[b]).wait()

        def out_start(gi, b):
            pltpu.async_copy(bufs[b], out_hbm.at[pl.ds(base + gi * CHUNK, CHUNK)], osems[b])

        def out_wait(gi, b):
            pltpu.make_async_copy(bufs[b], out_hbm.at[pl.ds(base + gi * CHUNK, CHUNK)], osems[b]).wait()

        for b in range(NBUF):
            gather(b, b)

        def outer(it, carry):
            for b in range(NBUF):
                gi = it * NBUF + b
                gather_wait(gi, b)

                def mul_body(r, c, b=b):
                    for rr in range(MUL_UNROLL):
                        row = r * MUL_UNROLL + rr
                        for j in range(EMB // 16):
                            sl = pl.ds(j * 16, 16)
                            bufs[b][row, sl] = bufs[b][row, sl] * SCALE
                    return c
                lax.fori_loop(0, CHUNK // MUL_UNROLL, mul_body, 0)

                out_start(gi, b)

                bt = (b + 2) % NBUF
                @pl.when(jnp.logical_and(gi >= 2, gi <= ngroups - 3))
                def _(gi=gi, bt=bt):
                    out_wait(gi - 2, bt)
                    gather(gi + 2, bt)
            return carry

        lax.fori_loop(0, iters, outer, 0)

        for b in range(NBUF):
            out_wait(ngroups - NBUF + b, b)

    return emb_kernel


def kernel(tokens, table):
    num_tokens = tokens.size
    tokens3 = tokens.reshape(NW, num_tokens // (NW * CHUNK), CHUNK).astype(jnp.int32)
    out = _build(num_tokens)(tokens3, table)
    return out.reshape(tokens.shape + (EMB,))
